# baseline (speedup 1.0000x reference)
import functools

import jax
import jax.numpy as jnp
from jax import lax
from jax.experimental import pallas as pl
from jax.experimental.pallas import tpu as pltpu
from jax.experimental.pallas import tpu_sc as plsc

H = 1024
E = 16
TOPK = 2
INTER = 512
T = 2048
P = T * TOPK
TILE = 256
PADDED = P + E * TILE
NTILES = PADDED // TILE

NW = 32
PAIRS_PER_W = P // NW
H2 = H // 2

_CH = 512


def _gate_body(x_ref, gw_ref, gb_ref, pos_ref, wts_ref, meta_ref):
    x = x_ref[...]
    gw = gw_ref[...]
    scores = lax.dot_general(x, gw, (((1,), (1,)), ((), ())),
                             preferred_element_type=jnp.float32)
    m = jnp.max(scores, axis=1, keepdims=True)
    ex = jnp.exp(scores - m)
    probs = ex / jnp.sum(ex, axis=1, keepdims=True)
    biased = probs + gb_ref[...]
    lane = lax.broadcasted_iota(jnp.int32, (T, E), 1)
    neg = jnp.float32(-1e30)
    m1 = jnp.max(biased, axis=1, keepdims=True)
    i1 = jnp.min(jnp.where(biased == m1, lane, E), axis=1, keepdims=True)
    rest = jnp.where(lane == i1, neg, biased)
    m2 = jnp.max(rest, axis=1, keepdims=True)
    i2 = jnp.min(jnp.where(rest == m2, lane, E), axis=1, keepdims=True)
    oh1 = (lane == i1).astype(jnp.float32)
    oh2 = (lane == i2).astype(jnp.float32)
    wts_ref[:, 0:1] = jnp.sum(probs * oh1, axis=1, keepdims=True)
    wts_ref[:, 1:2] = jnp.sum(probs * oh2, axis=1, keepdims=True)

    counts = (jnp.sum(oh1, axis=0, keepdims=True)
              + jnp.sum(oh2, axis=0, keepdims=True))
    pc = jnp.ceil(counts / TILE) * TILE
    r16 = lax.broadcasted_iota(jnp.int32, (E, E), 0)
    c16 = lax.broadcasted_iota(jnp.int32, (E, E), 1)
    strict_lt = (r16 < c16).astype(jnp.float32)
    off = lax.dot_general(pc, strict_lt, (((1,), (0,)), ((), ())),
                          precision=lax.Precision.HIGHEST,
                          preferred_element_type=jnp.float32)
    meta_ref[0:1, :] = counts
    meta_ref[1:2, :] = off
    meta_ref[2:8, :] = jnp.zeros((6, E), jnp.float32)

    rr = lax.broadcasted_iota(jnp.int32, (_CH, _CH), 0)
    cc = lax.broadcasted_iota(jnp.int32, (_CH, _CH), 1)
    tri = (cc < rr).astype(jnp.float32)
    carry = jnp.zeros((1, E), jnp.float32)
    for k, oh in ((0, oh1), (1, oh2)):
        for ci in range(T // _CH):
            blk = oh[ci * _CH:(ci + 1) * _CH, :]
            rank = lax.dot_general(tri, blk, (((1,), (0,)), ((), ())),
                                   precision=lax.Precision.HIGHEST,
                                   preferred_element_type=jnp.float32) + carry
            posb = jnp.sum((rank + off) * blk, axis=1, keepdims=True)
            pos_ref[ci * _CH:(ci + 1) * _CH, k:k + 1] = posb.astype(jnp.int32)
            carry = carry + jnp.sum(blk, axis=0, keepdims=True)


def _gate(xf, gate_w, gate_b):
    return pl.pallas_call(
        _gate_body,
        out_shape=(
            jax.ShapeDtypeStruct((T, TOPK), jnp.int32),
            jax.ShapeDtypeStruct((T, TOPK), jnp.float32),
            jax.ShapeDtypeStruct((8, E), jnp.float32),
        ),
    )(xf, gate_w, gate_b.reshape(1, E))


def _dispatch_body(pos_hbm, x_hbm, xs_hbm, pos_v, rows_v, sem):
    cid = lax.axis_index("c")
    sid = lax.axis_index("s")
    wid = sid * 2 + cid
    base = wid * PAIRS_PER_W
    tok0 = pl.multiple_of(jnp.bitwise_and(base, T - 1), PAIRS_PER_W)
    pltpu.sync_copy(pos_hbm.at[pl.ds(base, PAIRS_PER_W)], pos_v)
    pltpu.sync_copy(x_hbm.at[pl.ds(tok0, PAIRS_PER_W)], rows_v)
    pltpu.async_copy(rows_v, xs_hbm.at[pos_v], sem).wait()


def _dispatch(pos_flat, x2):
    mesh = plsc.VectorSubcoreMesh(core_axis_name="c", subcore_axis_name="s")
    fn = functools.partial(
        pl.kernel,
        out_type=jax.ShapeDtypeStruct((PADDED, H2), jnp.float32),
        mesh=mesh,
        scratch_types=[
            pltpu.VMEM((PAIRS_PER_W,), jnp.int32),
            pltpu.VMEM((PAIRS_PER_W, H2), jnp.float32),
            pltpu.SemaphoreType.DMA,
        ],
        compiler_params=pltpu.CompilerParams(needs_layout_passes=False),
    )(_dispatch_body)
    return fn(pos_flat, x2)


def _ffn_body(te_ref, xs_ref, w1_ref, w3_ref, w2_ref, b1_ref, b3_ref, b2_ref,
              out_ref):
    x = xs_ref[...]
    w1b = w1_ref[0].astype(jnp.bfloat16)
    w3b = w3_ref[0].astype(jnp.bfloat16)
    w2b = w2_ref[0].astype(jnp.bfloat16)
    h = lax.dot_general(x, w1b, (((1,), (1,)), ((), ())),
                        preferred_element_type=jnp.float32) + b1_ref[0]
    g = lax.dot_general(x, w3b, (((1,), (1,)), ((), ())),
                        preferred_element_type=jnp.float32) + b3_ref[0]
    a = h * g
    act = (a * jax.nn.sigmoid(a)).astype(jnp.bfloat16)
    o = lax.dot_general(act, w2b, (((1,), (1,)), ((), ())),
                        preferred_element_type=jnp.float32) + b2_ref[0]
    out_ref[...] = o.astype(jnp.bfloat16)


def _grouped_ffn(tile_expert, xs, w1, b1, w3, b3, w2, b2):
    grid_spec = pltpu.PrefetchScalarGridSpec(
        num_scalar_prefetch=1,
        grid=(NTILES,),
        in_specs=[
            pl.BlockSpec((TILE, H), lambda i, te: (i, 0)),
            pl.BlockSpec((1, INTER, H), lambda i, te: (te[i], 0, 0)),
            pl.BlockSpec((1, INTER, H), lambda i, te: (te[i], 0, 0)),
            pl.BlockSpec((1, H, INTER), lambda i, te: (te[i], 0, 0)),
            pl.BlockSpec((1, 1, INTER), lambda i, te: (te[i], 0, 0)),
            pl.BlockSpec((1, 1, INTER), lambda i, te: (te[i], 0, 0)),
            pl.BlockSpec((1, 1, H), lambda i, te: (te[i], 0, 0)),
        ],
        out_specs=pl.BlockSpec((TILE, H), lambda i, te: (i, 0)),
    )
    return pl.pallas_call(
        _ffn_body,
        grid_spec=grid_spec,
        out_shape=jax.ShapeDtypeStruct((PADDED, H), jnp.bfloat16),
    )(tile_expert, xs, w1, w3, w2, b1.reshape(E, 1, INTER),
      b3.reshape(E, 1, INTER), b2.reshape(E, 1, H))


def _gather_body(outs_hbm, pos_hbm, g_hbm, pos_v, rows_v, sem):
    cid = lax.axis_index("c")
    sid = lax.axis_index("s")
    wid = sid * 2 + cid
    base = wid * PAIRS_PER_W
    pltpu.sync_copy(pos_hbm.at[pl.ds(base, PAIRS_PER_W)], pos_v)
    pltpu.async_copy(outs_hbm.at[pos_v], rows_v, sem).wait()
    pltpu.sync_copy(rows_v, g_hbm.at[pl.ds(base, PAIRS_PER_W)])


def _gather_pairs(outs2, pos_flat):
    mesh = plsc.VectorSubcoreMesh(core_axis_name="c", subcore_axis_name="s")
    fn = functools.partial(
        pl.kernel,
        out_type=jax.ShapeDtypeStruct((P, H2), jnp.float32),
        mesh=mesh,
        scratch_types=[
            pltpu.VMEM((PAIRS_PER_W,), jnp.int32),
            pltpu.VMEM((PAIRS_PER_W, H2), jnp.float32),
            pltpu.SemaphoreType.DMA,
        ],
        compiler_params=pltpu.CompilerParams(needs_layout_passes=False),
    )(_gather_body)
    return fn(outs2, pos_flat)


def _sharedffn_body(x_ref, w1_ref, w3_ref, w2_ref, b1_ref, b3_ref, b2_ref,
                    z_ref):
    x = x_ref[...]
    h = lax.dot_general(x, w1_ref[...], (((1,), (1,)), ((), ())),
                        preferred_element_type=jnp.float32) + b1_ref[...]
    g = lax.dot_general(x, w3_ref[...], (((1,), (1,)), ((), ())),
                        preferred_element_type=jnp.float32) + b3_ref[...]
    a = h * g
    act = a * jax.nn.sigmoid(a)
    z_ref[...] = lax.dot_general(act, w2_ref[...], (((1,), (1,)), ((), ())),
                                 preferred_element_type=jnp.float32) + b2_ref[...]


def _shared_ffn(xf, sw1, sb1, sw3, sb3, sw2, sb2):
    blk = 512
    return pl.pallas_call(
        _sharedffn_body,
        grid=(T // blk,),
        in_specs=[
            pl.BlockSpec((blk, H), lambda i: (i, 0)),
            pl.BlockSpec((INTER, H), lambda i: (0, 0)),
            pl.BlockSpec((INTER, H), lambda i: (0, 0)),
            pl.BlockSpec((H, INTER), lambda i: (0, 0)),
            pl.BlockSpec((1, INTER), lambda i: (0, 0)),
            pl.BlockSpec((1, INTER), lambda i: (0, 0)),
            pl.BlockSpec((1, H), lambda i: (0, 0)),
        ],
        out_specs=pl.BlockSpec((blk, H), lambda i: (i, 0)),
        out_shape=jax.ShapeDtypeStruct((T, H), jnp.float32),
    )(xf, sw1, sw3, sw2, sb1.reshape(1, INTER), sb3.reshape(1, INTER),
      sb2.reshape(1, H))


def _combine_body(z_ref, g0_ref, g1_ref, wt_ref, y_ref):
    w = wt_ref[...]
    y_ref[...] = (z_ref[...]
                  + w[:, 0:1] * g0_ref[...].astype(jnp.float32)
                  + w[:, 1:2] * g1_ref[...].astype(jnp.float32))


def _combine(z, gpairs, wts_mat):
    blk = 512
    return pl.pallas_call(
        _combine_body,
        grid=(T // blk,),
        in_specs=[
            pl.BlockSpec((blk, H), lambda i: (i, 0)),
            pl.BlockSpec((blk, H), lambda i: (i, 0)),
            pl.BlockSpec((blk, H), lambda i: (i + T // blk, 0)),
            pl.BlockSpec((blk, TOPK), lambda i: (i, 0)),
        ],
        out_specs=pl.BlockSpec((blk, H), lambda i: (i, 0)),
        out_shape=jax.ShapeDtypeStruct((T, H), jnp.float32),
    )(z, gpairs, gpairs, wts_mat)


def kernel(x, gate_w, gate_b, w1, b1, w3, b3, w2, b2,
           sw1, sb1, sw3, sb3, sw2, sb2):
    xf = x.reshape(T, H)
    pos_mat, wts_mat, meta = _gate(xf, gate_w, gate_b)
    pos_flat = jnp.concatenate([pos_mat[:, 0], pos_mat[:, 1]])
    counts = meta[0]
    off = meta[1]
    ends = (off + jnp.ceil(counts / TILE) * TILE).astype(jnp.int32)
    tile_starts = jnp.arange(NTILES, dtype=jnp.int32) * TILE
    tile_expert = jnp.minimum(
        jnp.sum((ends[None, :] <= tile_starts[:, None]).astype(jnp.int32),
                axis=1), E - 1).astype(jnp.int32)

    x2 = lax.bitcast_convert_type(
        xf.astype(jnp.bfloat16).reshape(T, H2, 2), jnp.float32)
    xs2 = _dispatch(pos_flat, x2)
    xs = lax.bitcast_convert_type(xs2, jnp.bfloat16).reshape(PADDED, H)
    out_sorted = _grouped_ffn(tile_expert, xs, w1, b1, w3, b3, w2, b2)
    outs2 = lax.bitcast_convert_type(
        out_sorted.reshape(PADDED, H2, 2), jnp.float32)
    g2 = _gather_pairs(outs2, pos_flat)
    gpairs = lax.bitcast_convert_type(g2, jnp.bfloat16).reshape(P, H)
    z = _shared_ffn(xf, sw1, sb1, sw3, sb3, sw2, sb2)
    y = _combine(z, gpairs, wts_mat)
    return y.reshape(x.shape)


# speedup vs baseline: 3.4884x; 3.4884x over previous
import functools

import jax
import jax.numpy as jnp
from jax import lax
from jax.experimental import pallas as pl
from jax.experimental.pallas import tpu as pltpu
from jax.experimental.pallas import tpu_sc as plsc

H = 1024
E = 16
TOPK = 2
INTER = 512
T = 2048
P = T * TOPK
TILE = 256
PADDED = P + E * TILE
NTILES = PADDED // TILE

NW = 32
PAIRS_PER_W = P // NW
GCH = 64

_CH = 512


def _gate_body(x_ref, gw_ref, gb_ref, pos_ref, wts_ref, meta_ref):
    x = x_ref[...]
    gw = gw_ref[...]
    scores = lax.dot_general(x, gw, (((1,), (1,)), ((), ())),
                             preferred_element_type=jnp.float32)
    m = jnp.max(scores, axis=1, keepdims=True)
    ex = jnp.exp(scores - m)
    probs = ex / jnp.sum(ex, axis=1, keepdims=True)
    biased = probs + gb_ref[...]
    lane = lax.broadcasted_iota(jnp.int32, (T, E), 1)
    neg = jnp.float32(-1e30)
    m1 = jnp.max(biased, axis=1, keepdims=True)
    i1 = jnp.min(jnp.where(biased == m1, lane, E), axis=1, keepdims=True)
    rest = jnp.where(lane == i1, neg, biased)
    m2 = jnp.max(rest, axis=1, keepdims=True)
    i2 = jnp.min(jnp.where(rest == m2, lane, E), axis=1, keepdims=True)
    oh1 = (lane == i1).astype(jnp.float32)
    oh2 = (lane == i2).astype(jnp.float32)
    wts_ref[:, 0:1] = jnp.sum(probs * oh1, axis=1, keepdims=True)
    wts_ref[:, 1:2] = jnp.sum(probs * oh2, axis=1, keepdims=True)

    counts = (jnp.sum(oh1, axis=0, keepdims=True)
              + jnp.sum(oh2, axis=0, keepdims=True))
    pc = jnp.ceil(counts / TILE) * TILE
    r16 = lax.broadcasted_iota(jnp.int32, (E, E), 0)
    c16 = lax.broadcasted_iota(jnp.int32, (E, E), 1)
    strict_lt = (r16 < c16).astype(jnp.float32)
    off = lax.dot_general(pc, strict_lt, (((1,), (0,)), ((), ())),
                          precision=lax.Precision.HIGHEST,
                          preferred_element_type=jnp.float32)
    meta_ref[0:1, :] = counts
    meta_ref[1:2, :] = off
    meta_ref[2:8, :] = jnp.zeros((6, E), jnp.float32)

    rr = lax.broadcasted_iota(jnp.int32, (_CH, _CH), 0)
    cc = lax.broadcasted_iota(jnp.int32, (_CH, _CH), 1)
    tri = (cc < rr).astype(jnp.float32)
    carry = jnp.zeros((1, E), jnp.float32)
    for k, oh in ((0, oh1), (1, oh2)):
        for ci in range(T // _CH):
            blk = oh[ci * _CH:(ci + 1) * _CH, :]
            rank = lax.dot_general(tri, blk, (((1,), (0,)), ((), ())),
                                   precision=lax.Precision.HIGHEST,
                                   preferred_element_type=jnp.float32) + carry
            posb = jnp.sum((rank + off) * blk, axis=1, keepdims=True)
            pos_ref[ci * _CH:(ci + 1) * _CH, k:k + 1] = posb.astype(jnp.int32)
            carry = carry + jnp.sum(blk, axis=0, keepdims=True)


def _gate(xf, gate_w, gate_b):
    return pl.pallas_call(
        _gate_body,
        out_shape=(
            jax.ShapeDtypeStruct((T, TOPK), jnp.int32),
            jax.ShapeDtypeStruct((T, TOPK), jnp.float32),
            jax.ShapeDtypeStruct((8, E), jnp.float32),
        ),
    )(xf, gate_w, gate_b.reshape(1, E))


def _dispatch_body(pos_hbm, x_hbm, xs_hbm, pos_a, pos_b, rows_v, sem):
    cid = lax.axis_index("c")
    sid = lax.axis_index("s")
    wid = sid * 2 + cid
    base = wid * PAIRS_PER_W
    tok0 = pl.multiple_of(jnp.bitwise_and(base, T - 1), PAIRS_PER_W)
    pltpu.sync_copy(pos_hbm.at[pl.ds(base, GCH)], pos_a)
    pltpu.sync_copy(pos_hbm.at[pl.ds(base + GCH, GCH)], pos_b)
    pltpu.sync_copy(x_hbm.at[pl.ds(tok0, GCH)], rows_v)
    pltpu.async_copy(rows_v, xs_hbm.at[pos_a], sem).wait()
    pltpu.sync_copy(x_hbm.at[pl.ds(tok0 + GCH, GCH)], rows_v)
    pltpu.async_copy(rows_v, xs_hbm.at[pos_b], sem).wait()


def _dispatch(pos_flat, xf):
    mesh = plsc.VectorSubcoreMesh(core_axis_name="c", subcore_axis_name="s")
    fn = functools.partial(
        pl.kernel,
        out_type=jax.ShapeDtypeStruct((PADDED, H), jnp.float32),
        mesh=mesh,
        scratch_types=[
            pltpu.VMEM((GCH,), jnp.int32),
            pltpu.VMEM((GCH,), jnp.int32),
            pltpu.VMEM((GCH, H), jnp.float32),
            pltpu.SemaphoreType.DMA,
        ],
        compiler_params=pltpu.CompilerParams(needs_layout_passes=False),
    )(_dispatch_body)
    return fn(pos_flat, xf)


def _ffn_body(te_ref, xs_ref, w1_ref, w3_ref, w2_ref, b1_ref, b3_ref, b2_ref,
              out_ref):
    x = xs_ref[...].astype(jnp.bfloat16)
    w1b = w1_ref[0].astype(jnp.bfloat16)
    w3b = w3_ref[0].astype(jnp.bfloat16)
    w2b = w2_ref[0].astype(jnp.bfloat16)
    h = lax.dot_general(x, w1b, (((1,), (1,)), ((), ())),
                        preferred_element_type=jnp.float32) + b1_ref[0]
    g = lax.dot_general(x, w3b, (((1,), (1,)), ((), ())),
                        preferred_element_type=jnp.float32) + b3_ref[0]
    a = h * g
    act = (a * jax.nn.sigmoid(a)).astype(jnp.bfloat16)
    out_ref[...] = lax.dot_general(act, w2b, (((1,), (1,)), ((), ())),
                                   preferred_element_type=jnp.float32) + b2_ref[0]


def _grouped_ffn(tile_expert, xs, w1, b1, w3, b3, w2, b2):
    grid_spec = pltpu.PrefetchScalarGridSpec(
        num_scalar_prefetch=1,
        grid=(NTILES,),
        in_specs=[
            pl.BlockSpec((TILE, H), lambda i, te: (i, 0)),
            pl.BlockSpec((1, INTER, H), lambda i, te: (te[i], 0, 0)),
            pl.BlockSpec((1, INTER, H), lambda i, te: (te[i], 0, 0)),
            pl.BlockSpec((1, H, INTER), lambda i, te: (te[i], 0, 0)),
            pl.BlockSpec((1, 1, INTER), lambda i, te: (te[i], 0, 0)),
            pl.BlockSpec((1, 1, INTER), lambda i, te: (te[i], 0, 0)),
            pl.BlockSpec((1, 1, H), lambda i, te: (te[i], 0, 0)),
        ],
        out_specs=pl.BlockSpec((TILE, H), lambda i, te: (i, 0)),
    )
    return pl.pallas_call(
        _ffn_body,
        grid_spec=grid_spec,
        out_shape=jax.ShapeDtypeStruct((PADDED, H), jnp.float32),
    )(tile_expert, xs, w1, w3, w2, b1.reshape(E, 1, INTER),
      b3.reshape(E, 1, INTER), b2.reshape(E, 1, H))


def _gather_body(outs_hbm, pos_hbm, g_hbm, pos_v, rows_v, sem):
    cid = lax.axis_index("c")
    sid = lax.axis_index("s")
    wid = sid * 2 + cid
    base = wid * PAIRS_PER_W
    pltpu.sync_copy(pos_hbm.at[pl.ds(base, PAIRS_PER_W)], pos_v)
    for g in range(PAIRS_PER_W // GCH):
        o = g * GCH
        pltpu.async_copy(outs_hbm.at[pos_v.at[pl.ds(o, GCH)]], rows_v,
                         sem).wait()
        pltpu.sync_copy(rows_v, g_hbm.at[pl.ds(base + o, GCH)])


def _gather_pairs(out_sorted, pos_flat):
    mesh = plsc.VectorSubcoreMesh(core_axis_name="c", subcore_axis_name="s")
    fn = functools.partial(
        pl.kernel,
        out_type=jax.ShapeDtypeStruct((P, H), jnp.float32),
        mesh=mesh,
        scratch_types=[
            pltpu.VMEM((PAIRS_PER_W,), jnp.int32),
            pltpu.VMEM((GCH, H), jnp.float32),
            pltpu.SemaphoreType.DMA,
        ],
        compiler_params=pltpu.CompilerParams(needs_layout_passes=False),
    )(_gather_body)
    return fn(out_sorted, pos_flat)


def _sharedffn_body(x_ref, w1_ref, w3_ref, w2_ref, b1_ref, b3_ref, b2_ref,
                    z_ref):
    x = x_ref[...]
    h = lax.dot_general(x, w1_ref[...], (((1,), (1,)), ((), ())),
                        preferred_element_type=jnp.float32) + b1_ref[...]
    g = lax.dot_general(x, w3_ref[...], (((1,), (1,)), ((), ())),
                        preferred_element_type=jnp.float32) + b3_ref[...]
    a = h * g
    act = a * jax.nn.sigmoid(a)
    z_ref[...] = lax.dot_general(act, w2_ref[...], (((1,), (1,)), ((), ())),
                                 preferred_element_type=jnp.float32) + b2_ref[...]


def _shared_ffn(xf, sw1, sb1, sw3, sb3, sw2, sb2):
    blk = 512
    return pl.pallas_call(
        _sharedffn_body,
        grid=(T // blk,),
        in_specs=[
            pl.BlockSpec((blk, H), lambda i: (i, 0)),
            pl.BlockSpec((INTER, H), lambda i: (0, 0)),
            pl.BlockSpec((INTER, H), lambda i: (0, 0)),
            pl.BlockSpec((H, INTER), lambda i: (0, 0)),
            pl.BlockSpec((1, INTER), lambda i: (0, 0)),
            pl.BlockSpec((1, INTER), lambda i: (0, 0)),
            pl.BlockSpec((1, H), lambda i: (0, 0)),
        ],
        out_specs=pl.BlockSpec((blk, H), lambda i: (i, 0)),
        out_shape=jax.ShapeDtypeStruct((T, H), jnp.float32),
    )(xf, sw1, sw3, sw2, sb1.reshape(1, INTER), sb3.reshape(1, INTER),
      sb2.reshape(1, H))


def _combine_body(z_ref, g0_ref, g1_ref, wt_ref, y_ref):
    w = wt_ref[...]
    y_ref[...] = z_ref[...] + w[:, 0:1] * g0_ref[...] + w[:, 1:2] * g1_ref[...]


def _combine(z, gpairs, wts_mat):
    blk = 512
    return pl.pallas_call(
        _combine_body,
        grid=(T // blk,),
        in_specs=[
            pl.BlockSpec((blk, H), lambda i: (i, 0)),
            pl.BlockSpec((blk, H), lambda i: (i, 0)),
            pl.BlockSpec((blk, H), lambda i: (i + T // blk, 0)),
            pl.BlockSpec((blk, TOPK), lambda i: (i, 0)),
        ],
        out_specs=pl.BlockSpec((blk, H), lambda i: (i, 0)),
        out_shape=jax.ShapeDtypeStruct((T, H), jnp.float32),
    )(z, gpairs, gpairs, wts_mat)


def kernel(x, gate_w, gate_b, w1, b1, w3, b3, w2, b2,
           sw1, sb1, sw3, sb3, sw2, sb2):
    xf = x.reshape(T, H)
    pos_mat, wts_mat, meta = _gate(xf, gate_w, gate_b)
    pos_flat = jnp.concatenate([pos_mat[:, 0], pos_mat[:, 1]])
    counts = meta[0]
    off = meta[1]
    ends = (off + jnp.ceil(counts / TILE) * TILE).astype(jnp.int32)
    tile_starts = jnp.arange(NTILES, dtype=jnp.int32) * TILE
    tile_expert = jnp.minimum(
        jnp.sum((ends[None, :] <= tile_starts[:, None]).astype(jnp.int32),
                axis=1), E - 1).astype(jnp.int32)

    xs = _dispatch(pos_flat, xf)
    out_sorted = _grouped_ffn(tile_expert, xs, w1, b1, w3, b3, w2, b2)
    gpairs = _gather_pairs(out_sorted, pos_flat)
    z = _shared_ffn(xf, sw1, sb1, sw3, sb3, sw2, sb2)
    y = _combine(z, gpairs, wts_mat)
    return y.reshape(x.shape)


# speedup vs baseline: 3.5467x; 1.0167x over previous
import functools

import jax
import jax.numpy as jnp
from jax import lax
from jax.experimental import pallas as pl
from jax.experimental.pallas import tpu as pltpu
from jax.experimental.pallas import tpu_sc as plsc

H = 1024
E = 16
TOPK = 2
INTER = 512
T = 2048
P = T * TOPK
TILE = 256
PADDED = P + E * TILE
NTILES = PADDED // TILE

NW = 32
PAIRS_PER_W = P // NW
GCH = 64

_CH = 512


def _gate_body(x_ref, gw_ref, gb_ref, pos_ref, wts_ref, meta_ref):
    x = x_ref[...]
    gw = gw_ref[...]
    scores = lax.dot_general(x, gw, (((1,), (1,)), ((), ())),
                             preferred_element_type=jnp.float32)
    m = jnp.max(scores, axis=1, keepdims=True)
    ex = jnp.exp(scores - m)
    probs = ex / jnp.sum(ex, axis=1, keepdims=True)
    biased = probs + gb_ref[...]
    lane = lax.broadcasted_iota(jnp.int32, (T, E), 1)
    neg = jnp.float32(-1e30)
    m1 = jnp.max(biased, axis=1, keepdims=True)
    i1 = jnp.min(jnp.where(biased == m1, lane, E), axis=1, keepdims=True)
    rest = jnp.where(lane == i1, neg, biased)
    m2 = jnp.max(rest, axis=1, keepdims=True)
    i2 = jnp.min(jnp.where(rest == m2, lane, E), axis=1, keepdims=True)
    oh1 = (lane == i1).astype(jnp.float32)
    oh2 = (lane == i2).astype(jnp.float32)
    wts_ref[:, 0:1] = jnp.sum(probs * oh1, axis=1, keepdims=True)
    wts_ref[:, 1:2] = jnp.sum(probs * oh2, axis=1, keepdims=True)

    counts = (jnp.sum(oh1, axis=0, keepdims=True)
              + jnp.sum(oh2, axis=0, keepdims=True))
    pc = jnp.ceil(counts / TILE) * TILE
    r16 = lax.broadcasted_iota(jnp.int32, (E, E), 0)
    c16 = lax.broadcasted_iota(jnp.int32, (E, E), 1)
    strict_lt = (r16 < c16).astype(jnp.float32)
    off = lax.dot_general(pc, strict_lt, (((1,), (0,)), ((), ())),
                          preferred_element_type=jnp.float32)
    meta_ref[0:1, :] = counts
    meta_ref[1:2, :] = off
    meta_ref[2:8, :] = jnp.zeros((6, E), jnp.float32)

    rr = lax.broadcasted_iota(jnp.int32, (_CH, _CH), 0)
    cc = lax.broadcasted_iota(jnp.int32, (_CH, _CH), 1)
    tri = (cc < rr).astype(jnp.float32)
    carry = jnp.zeros((1, E), jnp.float32)
    for k, oh in ((0, oh1), (1, oh2)):
        for ci in range(T // _CH):
            blk = oh[ci * _CH:(ci + 1) * _CH, :]
            rank = lax.dot_general(tri, blk, (((1,), (0,)), ((), ())),
                                   preferred_element_type=jnp.float32) + carry
            posb = jnp.sum((rank + off) * blk, axis=1, keepdims=True)
            pos_ref[ci * _CH:(ci + 1) * _CH, k:k + 1] = posb.astype(jnp.int32)
            carry = carry + jnp.sum(blk, axis=0, keepdims=True)


def _gate(xf, gate_w, gate_b):
    return pl.pallas_call(
        _gate_body,
        out_shape=(
            jax.ShapeDtypeStruct((T, TOPK), jnp.int32),
            jax.ShapeDtypeStruct((T, TOPK), jnp.float32),
            jax.ShapeDtypeStruct((8, E), jnp.float32),
        ),
    )(xf, gate_w, gate_b.reshape(1, E))


def _dispatch_body(pos_hbm, x_hbm, xs_hbm, pos_a, pos_b, rows_v, sem):
    cid = lax.axis_index("c")
    sid = lax.axis_index("s")
    wid = sid * 2 + cid
    base = wid * PAIRS_PER_W
    tok0 = pl.multiple_of(jnp.bitwise_and(base, T - 1), PAIRS_PER_W)
    pltpu.sync_copy(pos_hbm.at[pl.ds(base, GCH)], pos_a)
    pltpu.sync_copy(pos_hbm.at[pl.ds(base + GCH, GCH)], pos_b)
    pltpu.sync_copy(x_hbm.at[pl.ds(tok0, GCH)], rows_v)
    pltpu.async_copy(rows_v, xs_hbm.at[pos_a], sem).wait()
    pltpu.sync_copy(x_hbm.at[pl.ds(tok0 + GCH, GCH)], rows_v)
    pltpu.async_copy(rows_v, xs_hbm.at[pos_b], sem).wait()


def _dispatch(pos_flat, xf):
    mesh = plsc.VectorSubcoreMesh(core_axis_name="c", subcore_axis_name="s")
    fn = functools.partial(
        pl.kernel,
        out_type=jax.ShapeDtypeStruct((PADDED, H), jnp.float32),
        mesh=mesh,
        scratch_types=[
            pltpu.VMEM((GCH,), jnp.int32),
            pltpu.VMEM((GCH,), jnp.int32),
            pltpu.VMEM((GCH, H), jnp.float32),
            pltpu.SemaphoreType.DMA,
        ],
        compiler_params=pltpu.CompilerParams(needs_layout_passes=False),
    )(_dispatch_body)
    return fn(pos_flat, xf)


def _ffn_body(te_ref, xs_ref, w1_ref, w3_ref, w2_ref, b1_ref, b3_ref, b2_ref,
              out_ref, w1s, w3s, w2s):
    i = pl.program_id(0)
    prev = te_ref[jnp.maximum(i - 1, 0)]

    @pl.when((i == 0) | (te_ref[i] != prev))
    def _cast_weights():
        w1s[...] = w1_ref[0].astype(jnp.bfloat16)
        w3s[...] = w3_ref[0].astype(jnp.bfloat16)
        w2s[...] = w2_ref[0].astype(jnp.bfloat16)

    x = xs_ref[...].astype(jnp.bfloat16)
    h = lax.dot_general(x, w1s[...], (((1,), (1,)), ((), ())),
                        preferred_element_type=jnp.float32) + b1_ref[0]
    g = lax.dot_general(x, w3s[...], (((1,), (1,)), ((), ())),
                        preferred_element_type=jnp.float32) + b3_ref[0]
    a = h * g
    act = (a * jax.nn.sigmoid(a)).astype(jnp.bfloat16)
    out_ref[...] = lax.dot_general(act, w2s[...], (((1,), (1,)), ((), ())),
                                   preferred_element_type=jnp.float32) + b2_ref[0]


def _grouped_ffn(tile_expert, xs, w1, b1, w3, b3, w2, b2):
    grid_spec = pltpu.PrefetchScalarGridSpec(
        num_scalar_prefetch=1,
        grid=(NTILES,),
        in_specs=[
            pl.BlockSpec((TILE, H), lambda i, te: (i, 0)),
            pl.BlockSpec((1, INTER, H), lambda i, te: (te[i], 0, 0)),
            pl.BlockSpec((1, INTER, H), lambda i, te: (te[i], 0, 0)),
            pl.BlockSpec((1, H, INTER), lambda i, te: (te[i], 0, 0)),
            pl.BlockSpec((1, 1, INTER), lambda i, te: (te[i], 0, 0)),
            pl.BlockSpec((1, 1, INTER), lambda i, te: (te[i], 0, 0)),
            pl.BlockSpec((1, 1, H), lambda i, te: (te[i], 0, 0)),
        ],
        out_specs=pl.BlockSpec((TILE, H), lambda i, te: (i, 0)),
        scratch_shapes=[
            pltpu.VMEM((INTER, H), jnp.bfloat16),
            pltpu.VMEM((INTER, H), jnp.bfloat16),
            pltpu.VMEM((H, INTER), jnp.bfloat16),
        ],
    )
    return pl.pallas_call(
        _ffn_body,
        grid_spec=grid_spec,
        out_shape=jax.ShapeDtypeStruct((PADDED, H), jnp.float32),
    )(tile_expert, xs, w1, w3, w2, b1.reshape(E, 1, INTER),
      b3.reshape(E, 1, INTER), b2.reshape(E, 1, H))


def _gather_body(outs_hbm, pos_hbm, g_hbm, pos_v, rows_v, sem):
    cid = lax.axis_index("c")
    sid = lax.axis_index("s")
    wid = sid * 2 + cid
    base = wid * PAIRS_PER_W
    pltpu.sync_copy(pos_hbm.at[pl.ds(base, PAIRS_PER_W)], pos_v)
    for g in range(PAIRS_PER_W // GCH):
        o = g * GCH
        pltpu.async_copy(outs_hbm.at[pos_v.at[pl.ds(o, GCH)]], rows_v,
                         sem).wait()
        pltpu.sync_copy(rows_v, g_hbm.at[pl.ds(base + o, GCH)])


def _gather_pairs(out_sorted, pos_flat):
    mesh = plsc.VectorSubcoreMesh(core_axis_name="c", subcore_axis_name="s")
    fn = functools.partial(
        pl.kernel,
        out_type=jax.ShapeDtypeStruct((P, H), jnp.float32),
        mesh=mesh,
        scratch_types=[
            pltpu.VMEM((PAIRS_PER_W,), jnp.int32),
            pltpu.VMEM((GCH, H), jnp.float32),
            pltpu.SemaphoreType.DMA,
        ],
        compiler_params=pltpu.CompilerParams(needs_layout_passes=False),
    )(_gather_body)
    return fn(out_sorted, pos_flat)


def _sharedffn_body(x_ref, w1_ref, w3_ref, w2_ref, b1_ref, b3_ref, b2_ref,
                    z_ref):
    x = x_ref[...]
    h = lax.dot_general(x, w1_ref[...], (((1,), (1,)), ((), ())),
                        preferred_element_type=jnp.float32) + b1_ref[...]
    g = lax.dot_general(x, w3_ref[...], (((1,), (1,)), ((), ())),
                        preferred_element_type=jnp.float32) + b3_ref[...]
    a = h * g
    act = a * jax.nn.sigmoid(a)
    z_ref[...] = lax.dot_general(act, w2_ref[...], (((1,), (1,)), ((), ())),
                                 preferred_element_type=jnp.float32) + b2_ref[...]


def _shared_ffn(xf, sw1, sb1, sw3, sb3, sw2, sb2):
    blk = 512
    return pl.pallas_call(
        _sharedffn_body,
        grid=(T // blk,),
        in_specs=[
            pl.BlockSpec((blk, H), lambda i: (i, 0)),
            pl.BlockSpec((INTER, H), lambda i: (0, 0)),
            pl.BlockSpec((INTER, H), lambda i: (0, 0)),
            pl.BlockSpec((H, INTER), lambda i: (0, 0)),
            pl.BlockSpec((1, INTER), lambda i: (0, 0)),
            pl.BlockSpec((1, INTER), lambda i: (0, 0)),
            pl.BlockSpec((1, H), lambda i: (0, 0)),
        ],
        out_specs=pl.BlockSpec((blk, H), lambda i: (i, 0)),
        out_shape=jax.ShapeDtypeStruct((T, H), jnp.float32),
    )(xf, sw1, sw3, sw2, sb1.reshape(1, INTER), sb3.reshape(1, INTER),
      sb2.reshape(1, H))


def _combine_body(z_ref, g0_ref, g1_ref, wt_ref, y_ref):
    w = wt_ref[...]
    y_ref[...] = z_ref[...] + w[:, 0:1] * g0_ref[...] + w[:, 1:2] * g1_ref[...]


def _combine(z, gpairs, wts_mat):
    blk = 512
    return pl.pallas_call(
        _combine_body,
        grid=(T // blk,),
        in_specs=[
            pl.BlockSpec((blk, H), lambda i: (i, 0)),
            pl.BlockSpec((blk, H), lambda i: (i, 0)),
            pl.BlockSpec((blk, H), lambda i: (i + T // blk, 0)),
            pl.BlockSpec((blk, TOPK), lambda i: (i, 0)),
        ],
        out_specs=pl.BlockSpec((blk, H), lambda i: (i, 0)),
        out_shape=jax.ShapeDtypeStruct((T, H), jnp.float32),
    )(z, gpairs, gpairs, wts_mat)


def kernel(x, gate_w, gate_b, w1, b1, w3, b3, w2, b2,
           sw1, sb1, sw3, sb3, sw2, sb2):
    xf = x.reshape(T, H)
    pos_mat, wts_mat, meta = _gate(xf, gate_w, gate_b)
    pos_flat = jnp.concatenate([pos_mat[:, 0], pos_mat[:, 1]])
    counts = meta[0]
    off = meta[1]
    ends = (off + jnp.ceil(counts / TILE) * TILE).astype(jnp.int32)
    tile_starts = jnp.arange(NTILES, dtype=jnp.int32) * TILE
    tile_expert = jnp.minimum(
        jnp.sum((ends[None, :] <= tile_starts[:, None]).astype(jnp.int32),
                axis=1), E - 1).astype(jnp.int32)

    xs = _dispatch(pos_flat, xf)
    out_sorted = _grouped_ffn(tile_expert, xs, w1, b1, w3, b3, w2, b2)
    gpairs = _gather_pairs(out_sorted, pos_flat)
    z = _shared_ffn(xf, sw1, sb1, sw3, sb3, sw2, sb2)
    y = _combine(z, gpairs, wts_mat)
    return y.reshape(x.shape)
